# baseline (speedup 1.0000x reference)
import functools

import numpy as np

import jax
import jax.numpy as jnp
from jax import lax
from jax.experimental import pallas as pl
from jax.experimental.pallas import tpu as pltpu
from jax.experimental.pallas import tpu_sc as plsc

N_NODES = 10000
N_EDGES = 320000
D = 128

NC = 2
NS = 16
NW = NC * NS
B = 128

EPW = -(-N_EDGES // NW)
NCH = -(-(-(-EPW // B)) // 2) * 2
EPW_P = NCH * B
E_PAD = NW * EPW_P
N_ACC = -(-(N_NODES + 1) // (NS * 8)) * (NS * 8)
ZROWS = N_ACC // NS


def _make_segsum():
    mesh = plsc.VectorSubcoreMesh(core_axis_name="c", subcore_axis_name="s",
                                  num_cores=NC, num_subcores=NS)

    @functools.partial(
        pl.kernel,
        out_type=jax.ShapeDtypeStruct((NC, N_ACC, D), jnp.float32),
        mesh=mesh,
        scratch_types=[
            pltpu.VMEM((NCH, B), jnp.int32),
            pltpu.VMEM((NCH, B), jnp.int32),
            pltpu.VMEM((B, D), jnp.float32),
            pltpu.SemaphoreType.DMA,
            pltpu.VMEM_SHARED((N_ACC, D), jnp.float32),
        ],
    )
    def segsum(x_hbm, src_hbm, dst_hbm, zeros_hbm, out_hbm,
               src_v, dst_v, rows_v, sem, acc):
        cid = lax.axis_index("c")
        sid = lax.axis_index("s")
        wid = sid * NC + cid

        pltpu.sync_copy(zeros_hbm.at[pl.ds(sid * ZROWS, ZROWS)],
                        acc.at[pl.ds(sid * ZROWS, ZROWS)])
        pltpu.sync_copy(src_hbm.at[wid], src_v)
        pltpu.sync_copy(dst_hbm.at[wid], dst_v)
        plsc.subcore_barrier()

        @pl.loop(0, NCH)
        def _chunk(j):
            pltpu.async_copy(x_hbm.at[src_v.at[j]], rows_v, sem).wait()
            pltpu.sync_copy(rows_v, acc.at[dst_v.at[j]], add=True)

        plsc.subcore_barrier()
        pltpu.sync_copy(acc.at[pl.ds(sid * ZROWS, ZROWS)],
                        out_hbm.at[cid, pl.ds(sid * ZROWS, ZROWS)])

    return segsum


_segsum = _make_segsum()


def _linrelu_body(seg_ref, w_ref, b_ref, out_ref):
    s = seg_ref[0] + seg_ref[1]
    out_ref[...] = jnp.maximum(
        jnp.dot(s, w_ref[...], preferred_element_type=jnp.float32)
        + b_ref[...], 0.0)


def _l2_pool_body(seg_ref, w2_ref, b2_ref, w3_ref, b3_ref, out_ref):
    i = pl.program_id(0)
    s = seg_ref[0] + seg_ref[1]
    h2 = jnp.maximum(
        jnp.dot(s, w2_ref[...], preferred_element_type=jnp.float32)
        + b2_ref[...], 0.0)
    colsum = jnp.sum(h2, axis=0, keepdims=True)

    @pl.when(i == 0)
    def _():
        out_ref[...] = colsum

    @pl.when(i > 0)
    def _():
        out_ref[...] = out_ref[...] + colsum

    @pl.when(i == pl.num_programs(0) - 1)
    def _():
        out_ref[...] = jnp.maximum(
            jnp.dot(out_ref[...], w3_ref[...],
                    preferred_element_type=jnp.float32)
            + b3_ref[...], 0.0)


_RB = 1000


def _linrelu(seg, w, b):
    grid = (N_NODES // _RB,)
    return pl.pallas_call(
        _linrelu_body,
        grid=grid,
        in_specs=[
            pl.BlockSpec((NC, _RB, D), lambda i: (0, i, 0)),
            pl.BlockSpec((D, D), lambda i: (0, 0)),
            pl.BlockSpec((1, D), lambda i: (0, 0)),
        ],
        out_specs=pl.BlockSpec((_RB, D), lambda i: (i, 0)),
        out_shape=jax.ShapeDtypeStruct((N_NODES, D), jnp.float32),
    )(seg, w, b)


def _l2_pool(seg, w2, b2, w3, b3):
    grid = (N_NODES // _RB,)
    return pl.pallas_call(
        _l2_pool_body,
        grid=grid,
        in_specs=[
            pl.BlockSpec((NC, _RB, D), lambda i: (0, i, 0)),
            pl.BlockSpec((D, D), lambda i: (0, 0)),
            pl.BlockSpec((1, D), lambda i: (0, 0)),
            pl.BlockSpec((D, D), lambda i: (0, 0)),
            pl.BlockSpec((1, D), lambda i: (0, 0)),
        ],
        out_specs=pl.BlockSpec((1, D), lambda i: (0, 0)),
        out_shape=jax.ShapeDtypeStruct((1, D), jnp.float32),
    )(seg, w2, b2, w3, b3)


def kernel(x, edge_index, W1, b1, W2, b2, W3, b3):
    src = edge_index[0].astype(jnp.int32)
    dst = edge_index[1].astype(jnp.int32)
    pad = E_PAD - N_EDGES
    src_p = jnp.concatenate([src, jnp.zeros((pad,), jnp.int32)]
                            ).reshape(NW, NCH, B)
    dst_p = jnp.concatenate([dst, jnp.full((pad,), N_NODES, jnp.int32)]
                            ).reshape(NW, NCH, B)
    zeros = np.zeros((N_ACC, D), np.float32)

    b1r = b1.reshape(1, D)
    b2r = b2.reshape(1, D)
    b3r = b3.reshape(1, D)

    seg1 = _segsum(x, src_p, dst_p, zeros)
    h = _linrelu(seg1, W1, b1r)
    seg2 = _segsum(h, src_p, dst_p, zeros)
    out = _l2_pool(seg2, W2, b2r, W3, b3r)
    return out


# speedup vs baseline: 1.4614x; 1.4614x over previous
import functools

import numpy as np

import jax
import jax.numpy as jnp
from jax import lax
from jax.experimental import pallas as pl
from jax.experimental.pallas import tpu as pltpu
from jax.experimental.pallas import tpu_sc as plsc

N_NODES = 10000
N_EDGES = 320000
D = 128

NC = 2
NS = 16
NW = NC * NS
B = 128

EPW = -(-N_EDGES // NW)
NCH = -(-EPW // B)
EPW_P = NCH * B
E_PAD = NW * EPW_P
N_ACC = -(-(N_NODES + 1) // (NS * 8)) * (NS * 8)
ZROWS = N_ACC // NS


def _make_segsum():
    mesh = plsc.VectorSubcoreMesh(core_axis_name="c", subcore_axis_name="s",
                                  num_cores=NC, num_subcores=NS)

    @functools.partial(
        pl.kernel,
        out_type=jax.ShapeDtypeStruct((NC, N_ACC, D), jnp.float32),
        mesh=mesh,
        scratch_types=[
            pltpu.VMEM((NCH, B), jnp.int32),
            pltpu.VMEM((NCH, B), jnp.int32),
            pltpu.VMEM((B, D), jnp.float32),
            pltpu.SemaphoreType.DMA,
            pltpu.VMEM_SHARED((N_ACC, D), jnp.float32),
        ],
    )
    def segsum(x_hbm, src_hbm, dst_hbm, zeros_hbm, out_hbm,
               src_v, dst_v, rows_v, sem, acc):
        cid = lax.axis_index("c")
        sid = lax.axis_index("s")
        wid = sid * NC + cid

        pltpu.sync_copy(zeros_hbm.at[pl.ds(sid * ZROWS, ZROWS)],
                        acc.at[pl.ds(sid * ZROWS, ZROWS)])
        pltpu.sync_copy(src_hbm.at[wid], src_v)
        pltpu.sync_copy(dst_hbm.at[wid], dst_v)
        plsc.subcore_barrier()

        @pl.loop(0, NCH)
        def _chunk(j):
            pltpu.async_copy(x_hbm.at[src_v.at[j]], rows_v, sem).wait()
            pltpu.sync_copy(rows_v, acc.at[dst_v.at[j]], add=True)

        plsc.subcore_barrier()
        pltpu.sync_copy(acc.at[pl.ds(sid * ZROWS, ZROWS)],
                        out_hbm.at[cid, pl.ds(sid * ZROWS, ZROWS)])

    return segsum


_segsum = _make_segsum()


def _linrelu_body(seg_ref, w_ref, b_ref, out_ref):
    s = seg_ref[0] + seg_ref[1]
    out_ref[...] = jnp.maximum(
        jnp.dot(s, w_ref[...], preferred_element_type=jnp.float32)
        + b_ref[...], 0.0)


def _l2_pool_body(seg_ref, w2_ref, b2_ref, w3_ref, b3_ref, out_ref):
    i = pl.program_id(0)
    s = seg_ref[0] + seg_ref[1]
    h2 = jnp.maximum(
        jnp.dot(s, w2_ref[...], preferred_element_type=jnp.float32)
        + b2_ref[...], 0.0)
    colsum = jnp.sum(h2, axis=0, keepdims=True)

    @pl.when(i == 0)
    def _():
        out_ref[...] = colsum

    @pl.when(i > 0)
    def _():
        out_ref[...] = out_ref[...] + colsum

    @pl.when(i == pl.num_programs(0) - 1)
    def _():
        out_ref[...] = jnp.maximum(
            jnp.dot(out_ref[...], w3_ref[...],
                    preferred_element_type=jnp.float32)
            + b3_ref[...], 0.0)


_RB = 1000


def _linrelu(seg, w, b):
    grid = (N_NODES // _RB,)
    return pl.pallas_call(
        _linrelu_body,
        grid=grid,
        in_specs=[
            pl.BlockSpec((NC, _RB, D), lambda i: (0, i, 0)),
            pl.BlockSpec((D, D), lambda i: (0, 0)),
            pl.BlockSpec((1, D), lambda i: (0, 0)),
        ],
        out_specs=pl.BlockSpec((_RB, D), lambda i: (i, 0)),
        out_shape=jax.ShapeDtypeStruct((N_NODES, D), jnp.float32),
    )(seg, w, b)


def _l2_pool(seg, w2, b2, w3, b3):
    grid = (N_NODES // _RB,)
    return pl.pallas_call(
        _l2_pool_body,
        grid=grid,
        in_specs=[
            pl.BlockSpec((NC, _RB, D), lambda i: (0, i, 0)),
            pl.BlockSpec((D, D), lambda i: (0, 0)),
            pl.BlockSpec((1, D), lambda i: (0, 0)),
            pl.BlockSpec((D, D), lambda i: (0, 0)),
            pl.BlockSpec((1, D), lambda i: (0, 0)),
        ],
        out_specs=pl.BlockSpec((1, D), lambda i: (0, 0)),
        out_shape=jax.ShapeDtypeStruct((1, D), jnp.float32),
    )(seg, w2, b2, w3, b3)


def kernel(x, edge_index, W1, b1, W2, b2, W3, b3):
    src = edge_index[0].astype(jnp.int32)
    dst = edge_index[1].astype(jnp.int32)
    pad = E_PAD - N_EDGES
    src_p = jnp.concatenate([src, jnp.zeros((pad,), jnp.int32)]
                            ).reshape(NW, NCH, B)
    dst_p = jnp.concatenate([dst, jnp.full((pad,), N_NODES, jnp.int32)]
                            ).reshape(NW, NCH, B)
    zeros = jnp.zeros((N_ACC, D), jnp.float32)

    b1r = b1.reshape(1, D)
    b2r = b2.reshape(1, D)
    b3r = b3.reshape(1, D)

    seg1 = _segsum(x, src_p, dst_p, zeros)
    h = _linrelu(seg1, W1, b1r)
    seg2 = _segsum(h, src_p, dst_p, zeros)
    out = _l2_pool(seg2, W2, b2r, W3, b3r)
    return out


# speedup vs baseline: 1.4626x; 1.0008x over previous
import functools

import numpy as np

import jax
import jax.numpy as jnp
from jax import lax
from jax.experimental import pallas as pl
from jax.experimental.pallas import tpu as pltpu
from jax.experimental.pallas import tpu_sc as plsc

N_NODES = 10000
N_EDGES = 320000
D = 128

NC = 2
NS = 16
NW = NC * NS
B = 128

EPW = -(-N_EDGES // NW)
NCH = -(-EPW // B)
EPW_P = NCH * B
E_PAD = NW * EPW_P
N_ACC = -(-(N_NODES + 1) // (NS * 8)) * (NS * 8)
ZROWS = N_ACC // NS


def _make_segsum():
    mesh = plsc.VectorSubcoreMesh(core_axis_name="c", subcore_axis_name="s",
                                  num_cores=NC, num_subcores=NS)

    @functools.partial(
        pl.kernel,
        out_type=jax.ShapeDtypeStruct((NC, N_ACC, D), jnp.float32),
        mesh=mesh,
        scratch_types=[
            pltpu.VMEM((NCH, B), jnp.int32),
            pltpu.VMEM((NCH, B), jnp.int32),
            pltpu.VMEM((B, D), jnp.float32),
            pltpu.SemaphoreType.DMA,
            pltpu.VMEM_SHARED((N_ACC, D), jnp.float32),
        ],
    )
    def segsum(x_hbm, src_hbm, dst_hbm, zeros_hbm, out_hbm,
               src_v, dst_v, rows_v, sem, acc):
        cid = lax.axis_index("c")
        sid = lax.axis_index("s")
        wid = sid * NC + cid

        pltpu.sync_copy(zeros_hbm.at[pl.ds(sid * ZROWS, ZROWS)],
                        acc.at[pl.ds(sid * ZROWS, ZROWS)])
        pltpu.sync_copy(src_hbm.at[wid], src_v)
        pltpu.sync_copy(dst_hbm.at[wid], dst_v)
        plsc.subcore_barrier()

        @pl.loop(0, NCH)
        def _chunk(j):
            pltpu.async_copy(x_hbm.at[src_v.at[j]], rows_v, sem).wait()
            pltpu.sync_copy(rows_v, acc.at[dst_v.at[j]], add=True)

        plsc.subcore_barrier()
        pltpu.sync_copy(acc.at[pl.ds(sid * ZROWS, ZROWS)],
                        out_hbm.at[cid, pl.ds(sid * ZROWS, ZROWS)])

    return segsum


_segsum = _make_segsum()


def _linrelu_body(seg_ref, w_ref, b_ref, out_ref):
    s = seg_ref[0] + seg_ref[1]
    out_ref[...] = jnp.maximum(
        jnp.dot(s.astype(jnp.bfloat16), w_ref[...].astype(jnp.bfloat16),
                preferred_element_type=jnp.float32)
        + b_ref[...], 0.0)


def _l2_pool_body(seg_ref, w2_ref, b2_ref, w3_ref, b3_ref, out_ref):
    i = pl.program_id(0)
    s = seg_ref[0] + seg_ref[1]
    h2 = jnp.maximum(
        jnp.dot(s.astype(jnp.bfloat16), w2_ref[...].astype(jnp.bfloat16),
                preferred_element_type=jnp.float32)
        + b2_ref[...], 0.0)
    colsum = jnp.sum(h2, axis=0, keepdims=True)

    @pl.when(i == 0)
    def _():
        out_ref[...] = colsum

    @pl.when(i > 0)
    def _():
        out_ref[...] = out_ref[...] + colsum

    @pl.when(i == pl.num_programs(0) - 1)
    def _():
        out_ref[...] = jnp.maximum(
            jnp.dot(out_ref[...], w3_ref[...],
                    preferred_element_type=jnp.float32)
            + b3_ref[...], 0.0)


_RB = 1000


def _linrelu(seg, w, b):
    grid = (N_NODES // _RB,)
    return pl.pallas_call(
        _linrelu_body,
        grid=grid,
        in_specs=[
            pl.BlockSpec((NC, _RB, D), lambda i: (0, i, 0)),
            pl.BlockSpec((D, D), lambda i: (0, 0)),
            pl.BlockSpec((1, D), lambda i: (0, 0)),
        ],
        out_specs=pl.BlockSpec((_RB, D), lambda i: (i, 0)),
        out_shape=jax.ShapeDtypeStruct((N_NODES, D), jnp.float32),
    )(seg, w, b)


def _l2_pool(seg, w2, b2, w3, b3):
    grid = (N_NODES // _RB,)
    return pl.pallas_call(
        _l2_pool_body,
        grid=grid,
        in_specs=[
            pl.BlockSpec((NC, _RB, D), lambda i: (0, i, 0)),
            pl.BlockSpec((D, D), lambda i: (0, 0)),
            pl.BlockSpec((1, D), lambda i: (0, 0)),
            pl.BlockSpec((D, D), lambda i: (0, 0)),
            pl.BlockSpec((1, D), lambda i: (0, 0)),
        ],
        out_specs=pl.BlockSpec((1, D), lambda i: (0, 0)),
        out_shape=jax.ShapeDtypeStruct((1, D), jnp.float32),
    )(seg, w2, b2, w3, b3)


def kernel(x, edge_index, W1, b1, W2, b2, W3, b3):
    src = edge_index[0].astype(jnp.int32)
    dst = edge_index[1].astype(jnp.int32)
    pad = E_PAD - N_EDGES
    src_p = jnp.concatenate([src, jnp.zeros((pad,), jnp.int32)]
                            ).reshape(NW, NCH, B)
    dst_p = jnp.concatenate([dst, jnp.full((pad,), N_NODES, jnp.int32)]
                            ).reshape(NW, NCH, B)
    zeros = jnp.zeros((N_ACC, D), jnp.float32)

    b1r = b1.reshape(1, D)
    b2r = b2.reshape(1, D)
    b3r = b3.reshape(1, D)

    seg1 = _segsum(x, src_p, dst_p, zeros)
    h = _linrelu(seg1, W1, b1r)
    seg2 = _segsum(h, src_p, dst_p, zeros)
    out = _l2_pool(seg2, W2, b2r, W3, b3r)
    return out


# speedup vs baseline: 1.9043x; 1.3020x over previous
import functools

import numpy as np

import jax
import jax.numpy as jnp
from jax import lax
from jax.experimental import pallas as pl
from jax.experimental.pallas import tpu as pltpu
from jax.experimental.pallas import tpu_sc as plsc

N_NODES = 10000
N_EDGES = 320000
D = 128

NC = 2
NS = 16
NW = NC * NS
B = 128

FAST_CID = 0
CT = -(-N_EDGES // B)
CPP = -(-CT // NS)
NCH_F = -(-CPP * 304 // (304 + 185))
NCH_S = CPP - NCH_F
E_PAD = NS * CPP * B
N_ACC = -(-(N_NODES + 1) // (NS * 8)) * (NS * 8)
ZROWS = N_ACC // NS


def _make_segsum():
    mesh = plsc.VectorSubcoreMesh(core_axis_name="c", subcore_axis_name="s",
                                  num_cores=NC, num_subcores=NS)

    @functools.partial(
        pl.kernel,
        out_type=jax.ShapeDtypeStruct((NC, N_ACC, D), jnp.float32),
        mesh=mesh,
        scratch_types=[
            pltpu.VMEM((NCH_F, B), jnp.int32),
            pltpu.VMEM((NCH_F, B), jnp.int32),
            pltpu.VMEM((B, D), jnp.float32),
            pltpu.SemaphoreType.DMA,
            pltpu.VMEM_SHARED((N_ACC, D), jnp.float32),
        ],
    )
    def segsum(x_hbm, src_hbm, dst_hbm, zeros_hbm, out_hbm,
               src_v, dst_v, rows_v, sem, acc):
        cid = lax.axis_index("c")
        sid = lax.axis_index("s")
        wid = sid * NC + cid

        pltpu.sync_copy(zeros_hbm.at[pl.ds(sid * ZROWS, ZROWS)],
                        acc.at[pl.ds(sid * ZROWS, ZROWS)])
        pltpu.sync_copy(src_hbm.at[wid], src_v)
        pltpu.sync_copy(dst_hbm.at[wid], dst_v)
        plsc.subcore_barrier()

        nch = lax.select(cid == FAST_CID, NCH_F, NCH_S)

        @pl.loop(0, nch)
        def _chunk(j):
            pltpu.async_copy(x_hbm.at[src_v.at[j]], rows_v, sem).wait()
            pltpu.sync_copy(rows_v, acc.at[dst_v.at[j]], add=True)

        plsc.subcore_barrier()
        pltpu.sync_copy(acc.at[pl.ds(sid * ZROWS, ZROWS)],
                        out_hbm.at[cid, pl.ds(sid * ZROWS, ZROWS)])

    return segsum


_segsum = _make_segsum()


def _linrelu_body(seg_ref, w_ref, b_ref, out_ref):
    s = seg_ref[0] + seg_ref[1]
    out_ref[...] = jnp.maximum(
        jnp.dot(s.astype(jnp.bfloat16), w_ref[...].astype(jnp.bfloat16),
                preferred_element_type=jnp.float32)
        + b_ref[...], 0.0)


def _l2_pool_body(seg_ref, w2_ref, b2_ref, w3_ref, b3_ref, out_ref):
    i = pl.program_id(0)
    s = seg_ref[0] + seg_ref[1]
    h2 = jnp.maximum(
        jnp.dot(s.astype(jnp.bfloat16), w2_ref[...].astype(jnp.bfloat16),
                preferred_element_type=jnp.float32)
        + b2_ref[...], 0.0)
    colsum = jnp.sum(h2, axis=0, keepdims=True)

    @pl.when(i == 0)
    def _():
        out_ref[...] = colsum

    @pl.when(i > 0)
    def _():
        out_ref[...] = out_ref[...] + colsum

    @pl.when(i == pl.num_programs(0) - 1)
    def _():
        out_ref[...] = jnp.maximum(
            jnp.dot(out_ref[...], w3_ref[...],
                    preferred_element_type=jnp.float32)
            + b3_ref[...], 0.0)


_RB = 1000


def _linrelu(seg, w, b):
    grid = (N_NODES // _RB,)
    return pl.pallas_call(
        _linrelu_body,
        grid=grid,
        in_specs=[
            pl.BlockSpec((NC, _RB, D), lambda i: (0, i, 0)),
            pl.BlockSpec((D, D), lambda i: (0, 0)),
            pl.BlockSpec((1, D), lambda i: (0, 0)),
        ],
        out_specs=pl.BlockSpec((_RB, D), lambda i: (i, 0)),
        out_shape=jax.ShapeDtypeStruct((N_NODES, D), jnp.float32),
    )(seg, w, b)


def _l2_pool(seg, w2, b2, w3, b3):
    grid = (N_NODES // _RB,)
    return pl.pallas_call(
        _l2_pool_body,
        grid=grid,
        in_specs=[
            pl.BlockSpec((NC, _RB, D), lambda i: (0, i, 0)),
            pl.BlockSpec((D, D), lambda i: (0, 0)),
            pl.BlockSpec((1, D), lambda i: (0, 0)),
            pl.BlockSpec((D, D), lambda i: (0, 0)),
            pl.BlockSpec((1, D), lambda i: (0, 0)),
        ],
        out_specs=pl.BlockSpec((1, D), lambda i: (0, 0)),
        out_shape=jax.ShapeDtypeStruct((1, D), jnp.float32),
    )(seg, w2, b2, w3, b3)


def _layout_edges(e, fill):
    pad = E_PAD - N_EDGES
    ep = jnp.concatenate([e, jnp.full((pad,), fill, jnp.int32)]
                         ).reshape(NS, CPP * B)
    fast = ep[:, :NCH_F * B].reshape(NS, 1, NCH_F, B)
    slow = jnp.concatenate(
        [ep[:, NCH_F * B:],
         jnp.full((NS, (NCH_F - NCH_S) * B), fill, jnp.int32)],
        axis=1).reshape(NS, 1, NCH_F, B)
    pair = [fast, slow] if FAST_CID == 0 else [slow, fast]
    return jnp.concatenate(pair, axis=1).reshape(NW, NCH_F, B)


def kernel(x, edge_index, W1, b1, W2, b2, W3, b3):
    src = edge_index[0].astype(jnp.int32)
    dst = edge_index[1].astype(jnp.int32)
    src_p = _layout_edges(src, 0)
    dst_p = _layout_edges(dst, N_NODES)
    zeros = jnp.zeros((N_ACC, D), jnp.float32)

    b1r = b1.reshape(1, D)
    b2r = b2.reshape(1, D)
    b3r = b3.reshape(1, D)

    seg1 = _segsum(x, src_p, dst_p, zeros)
    h = _linrelu(seg1, W1, b1r)
    seg2 = _segsum(h, src_p, dst_p, zeros)
    out = _l2_pool(seg2, W2, b2r, W3, b3r)
    return out


# speedup vs baseline: 1.9045x; 1.0001x over previous
import functools

import numpy as np

import jax
import jax.numpy as jnp
from jax import lax
from jax.experimental import pallas as pl
from jax.experimental.pallas import tpu as pltpu
from jax.experimental.pallas import tpu_sc as plsc

N_NODES = 10000
N_EDGES = 320000
D = 128

NC = 2
NS = 16
NW = NC * NS
B = 128

FAST_CID = 0
CT = -(-N_EDGES // B)
CPP = -(-CT // NS)
NCH_F = -(-CPP * 304 // (304 + 185))
NCH_S = CPP - NCH_F
E_PAD = NS * CPP * B
N_ACC = -(-(N_NODES + 1) // (NS * 16)) * (NS * 16)
ZROWS = N_ACC // NS


def _make_segsum():
    mesh = plsc.VectorSubcoreMesh(core_axis_name="c", subcore_axis_name="s",
                                  num_cores=NC, num_subcores=NS)

    @functools.partial(
        pl.kernel,
        out_type=jax.ShapeDtypeStruct((NC, N_ACC, D), jnp.float32),
        mesh=mesh,
        scratch_types=[
            pltpu.VMEM((NCH_F, B), jnp.int32),
            pltpu.VMEM((NCH_F, B), jnp.int32),
            pltpu.VMEM((B, D), jnp.float32),
            pltpu.SemaphoreType.DMA,
            pltpu.VMEM_SHARED((N_ACC, D), jnp.float32),
        ],
    )
    def segsum(x_hbm, src_hbm, dst_hbm, zeros_hbm, out_hbm,
               src_v, dst_v, rows_v, sem, acc):
        cid = lax.axis_index("c")
        sid = lax.axis_index("s")
        wid = sid * NC + cid

        pltpu.sync_copy(zeros_hbm.at[pl.ds(sid * ZROWS, ZROWS)],
                        acc.at[pl.ds(sid * ZROWS, ZROWS)])
        pltpu.sync_copy(src_hbm.at[wid], src_v)
        pltpu.sync_copy(dst_hbm.at[wid], dst_v)
        plsc.subcore_barrier()

        nch = lax.select(cid == FAST_CID, NCH_F, NCH_S)

        @pl.loop(0, nch)
        def _chunk(j):
            pltpu.async_copy(x_hbm.at[src_v.at[j]], rows_v, sem).wait()
            pltpu.sync_copy(rows_v, acc.at[dst_v.at[j]], add=True)

        plsc.subcore_barrier()
        pltpu.sync_copy(acc.at[pl.ds(sid * ZROWS, ZROWS)],
                        out_hbm.at[cid, pl.ds(sid * ZROWS, ZROWS)])

    return segsum


_segsum = _make_segsum()


def _linrelu_body(seg_ref, w_ref, b_ref, out_ref):
    s = seg_ref[0] + seg_ref[1]
    out_ref[...] = jnp.maximum(
        jnp.dot(s.astype(jnp.bfloat16), w_ref[...].astype(jnp.bfloat16),
                preferred_element_type=jnp.float32)
        + b_ref[...], 0.0)


def _l2_pool_body(seg_ref, w2_ref, b2_ref, w3_ref, b3_ref, out_ref):
    i = pl.program_id(0)
    s = seg_ref[0] + seg_ref[1]
    h2 = jnp.maximum(
        jnp.dot(s.astype(jnp.bfloat16), w2_ref[...].astype(jnp.bfloat16),
                preferred_element_type=jnp.float32)
        + b2_ref[...], 0.0)
    colsum = jnp.sum(h2, axis=0, keepdims=True)

    @pl.when(i == 0)
    def _():
        out_ref[...] = colsum

    @pl.when(i > 0)
    def _():
        out_ref[...] = out_ref[...] + colsum

    @pl.when(i == pl.num_programs(0) - 1)
    def _():
        out_ref[...] = jnp.maximum(
            jnp.dot(out_ref[...], w3_ref[...],
                    preferred_element_type=jnp.float32)
            + b3_ref[...], 0.0)


_RB = 1000


def _linrelu(seg, w, b):
    grid = (N_NODES // _RB,)
    return pl.pallas_call(
        _linrelu_body,
        grid=grid,
        in_specs=[
            pl.BlockSpec((NC, _RB, D), lambda i: (0, i, 0)),
            pl.BlockSpec((D, D), lambda i: (0, 0)),
            pl.BlockSpec((1, D), lambda i: (0, 0)),
        ],
        out_specs=pl.BlockSpec((_RB, D), lambda i: (i, 0)),
        out_shape=jax.ShapeDtypeStruct((N_NODES, D), jnp.float32),
    )(seg, w, b)


def _l2_pool(seg, w2, b2, w3, b3):
    grid = (N_NODES // _RB,)
    return pl.pallas_call(
        _l2_pool_body,
        grid=grid,
        in_specs=[
            pl.BlockSpec((NC, _RB, D), lambda i: (0, i, 0)),
            pl.BlockSpec((D, D), lambda i: (0, 0)),
            pl.BlockSpec((1, D), lambda i: (0, 0)),
            pl.BlockSpec((D, D), lambda i: (0, 0)),
            pl.BlockSpec((1, D), lambda i: (0, 0)),
        ],
        out_specs=pl.BlockSpec((1, D), lambda i: (0, 0)),
        out_shape=jax.ShapeDtypeStruct((1, D), jnp.float32),
    )(seg, w2, b2, w3, b3)


def _layout_edges(e, fill):
    pad = E_PAD - N_EDGES
    ep = jnp.concatenate([e, jnp.full((pad,), fill, jnp.int32)]
                         ).reshape(NS, CPP * B)
    fast = ep[:, :NCH_F * B].reshape(NS, 1, NCH_F, B)
    slow = jnp.concatenate(
        [ep[:, NCH_F * B:],
         jnp.full((NS, (NCH_F - NCH_S) * B), fill, jnp.int32)],
        axis=1).reshape(NS, 1, NCH_F, B)
    pair = [fast, slow] if FAST_CID == 0 else [slow, fast]
    return jnp.concatenate(pair, axis=1).reshape(NW, NCH_F, B)


def kernel(x, edge_index, W1, b1, W2, b2, W3, b3):
    src = edge_index[0].astype(jnp.int32)
    dst = edge_index[1].astype(jnp.int32)
    src_p = _layout_edges(src, 0)
    dst_p = _layout_edges(dst, N_NODES)
    zeros = jnp.zeros((N_ACC, D), jnp.float32)

    b1r = b1.reshape(1, D)
    b2r = b2.reshape(1, D)
    b3r = b3.reshape(1, D)

    seg1 = _segsum(x, src_p, dst_p, zeros)
    h = _linrelu(seg1, W1, b1r)
    seg2 = _segsum(h, src_p, dst_p, zeros)
    out = _l2_pool(seg2, W2, b2r, W3, b3r)
    return out


# speedup vs baseline: 1.9586x; 1.0284x over previous
import functools

import numpy as np

import jax
import jax.numpy as jnp
from jax import lax
from jax.experimental import pallas as pl
from jax.experimental.pallas import tpu as pltpu
from jax.experimental.pallas import tpu_sc as plsc

N_NODES = 10000
N_EDGES = 320000
D = 128

NC = 2
NS = 16
NW = NC * NS
B = 128

FAST_CID = 0
CT = -(-N_EDGES // B)
CPP = -(-CT // NS)
NCH_F = -(-CPP * 187 // (187 + 129))
NCH_S = CPP - NCH_F
E_PAD = NS * CPP * B
N_ACC = -(-(N_NODES + 1) // (NS * 16)) * (NS * 16)
ZROWS = N_ACC // NS


def _make_segsum():
    mesh = plsc.VectorSubcoreMesh(core_axis_name="c", subcore_axis_name="s",
                                  num_cores=NC, num_subcores=NS)

    @functools.partial(
        pl.kernel,
        out_type=jax.ShapeDtypeStruct((NC, N_ACC, D), jnp.float32),
        mesh=mesh,
        scratch_types=[
            pltpu.VMEM((NCH_F, B), jnp.int32),
            pltpu.VMEM((NCH_F, B), jnp.int32),
            pltpu.VMEM((B, D), jnp.float32),
            pltpu.SemaphoreType.DMA,
            pltpu.VMEM_SHARED((N_ACC, D), jnp.float32),
        ],
    )
    def segsum(x_hbm, src_hbm, dst_hbm, out_hbm,
               src_v, dst_v, rows_v, sem, acc):
        cid = lax.axis_index("c")
        sid = lax.axis_index("s")
        wid = sid * NC + cid

        zv = jnp.zeros((16,), jnp.float32)

        @pl.loop(0, B)
        def _zrow(r):
            for c in range(D // 16):
                rows_v[r, pl.ds(c * 16, 16)] = zv

        for k in range(ZROWS // B):
            pltpu.sync_copy(rows_v, acc.at[pl.ds(sid * ZROWS + k * B, B)])
        pltpu.sync_copy(src_hbm.at[wid], src_v)
        pltpu.sync_copy(dst_hbm.at[wid], dst_v)
        plsc.subcore_barrier()

        nch = lax.select(cid == FAST_CID, NCH_F, NCH_S)

        @pl.loop(0, nch)
        def _chunk(j):
            pltpu.async_copy(x_hbm.at[src_v.at[j]], rows_v, sem).wait()
            pltpu.sync_copy(rows_v, acc.at[dst_v.at[j]], add=True)

        plsc.subcore_barrier()
        pltpu.sync_copy(acc.at[pl.ds(sid * ZROWS, ZROWS)],
                        out_hbm.at[cid, pl.ds(sid * ZROWS, ZROWS)])

    return segsum


_segsum = _make_segsum()


def _linrelu_body(seg_ref, w_ref, b_ref, out_ref):
    s = seg_ref[0] + seg_ref[1]
    out_ref[...] = jnp.maximum(
        jnp.dot(s.astype(jnp.bfloat16), w_ref[...].astype(jnp.bfloat16),
                preferred_element_type=jnp.float32)
        + b_ref[...], 0.0)


def _l2_pool_body(seg_ref, w2_ref, b2_ref, w3_ref, b3_ref, out_ref):
    i = pl.program_id(0)
    s = seg_ref[0] + seg_ref[1]
    h2 = jnp.maximum(
        jnp.dot(s.astype(jnp.bfloat16), w2_ref[...].astype(jnp.bfloat16),
                preferred_element_type=jnp.float32)
        + b2_ref[...], 0.0)
    colsum = jnp.sum(h2, axis=0, keepdims=True)

    @pl.when(i == 0)
    def _():
        out_ref[...] = colsum

    @pl.when(i > 0)
    def _():
        out_ref[...] = out_ref[...] + colsum

    @pl.when(i == pl.num_programs(0) - 1)
    def _():
        out_ref[...] = jnp.maximum(
            jnp.dot(out_ref[...], w3_ref[...],
                    preferred_element_type=jnp.float32)
            + b3_ref[...], 0.0)


_RB = 1000


def _linrelu(seg, w, b):
    grid = (N_NODES // _RB,)
    return pl.pallas_call(
        _linrelu_body,
        grid=grid,
        in_specs=[
            pl.BlockSpec((NC, _RB, D), lambda i: (0, i, 0)),
            pl.BlockSpec((D, D), lambda i: (0, 0)),
            pl.BlockSpec((1, D), lambda i: (0, 0)),
        ],
        out_specs=pl.BlockSpec((_RB, D), lambda i: (i, 0)),
        out_shape=jax.ShapeDtypeStruct((N_NODES, D), jnp.float32),
    )(seg, w, b)


def _l2_pool(seg, w2, b2, w3, b3):
    grid = (N_NODES // _RB,)
    return pl.pallas_call(
        _l2_pool_body,
        grid=grid,
        in_specs=[
            pl.BlockSpec((NC, _RB, D), lambda i: (0, i, 0)),
            pl.BlockSpec((D, D), lambda i: (0, 0)),
            pl.BlockSpec((1, D), lambda i: (0, 0)),
            pl.BlockSpec((D, D), lambda i: (0, 0)),
            pl.BlockSpec((1, D), lambda i: (0, 0)),
        ],
        out_specs=pl.BlockSpec((1, D), lambda i: (0, 0)),
        out_shape=jax.ShapeDtypeStruct((1, D), jnp.float32),
    )(seg, w2, b2, w3, b3)


def _layout_edges(e, fill):
    pad = E_PAD - N_EDGES
    ep = jnp.concatenate([e, jnp.full((pad,), fill, jnp.int32)]
                         ).reshape(NS, CPP * B)
    fast = ep[:, :NCH_F * B].reshape(NS, 1, NCH_F, B)
    slow = jnp.concatenate(
        [ep[:, NCH_F * B:],
         jnp.full((NS, (NCH_F - NCH_S) * B), fill, jnp.int32)],
        axis=1).reshape(NS, 1, NCH_F, B)
    pair = [fast, slow] if FAST_CID == 0 else [slow, fast]
    return jnp.concatenate(pair, axis=1).reshape(NW, NCH_F, B)


def kernel(x, edge_index, W1, b1, W2, b2, W3, b3):
    src = edge_index[0].astype(jnp.int32)
    dst = edge_index[1].astype(jnp.int32)
    src_p = _layout_edges(src, 0)
    dst_p = _layout_edges(dst, N_NODES)

    b1r = b1.reshape(1, D)
    b2r = b2.reshape(1, D)
    b3r = b3.reshape(1, D)

    seg1 = _segsum(x, src_p, dst_p)
    h = _linrelu(seg1, W1, b1r)
    seg2 = _segsum(h, src_p, dst_p)
    out = _l2_pool(seg2, W2, b2r, W3, b3r)
    return out
